# baseline (speedup 1.0000x reference)
import functools

import jax
import jax.numpy as jnp
from jax import lax
from jax.experimental import pallas as pl
from jax.experimental.pallas import tpu as pltpu
from jax.experimental.pallas import tpu_sc as plsc

VOCAB = 1000000
B = 16384
D = 32
N_NEG = 20
L = 16
NC = 2
NS = 16
NW = NC * NS
BPW = B // NW

VC = 800
NCHUNK = VOCAB // VC
NK = (NCHUNK + NW - 1) // NW
DP = D + 1


def _tr_body(wht_hbm, wot_hbm, wh_out, wo_out,
             in_h, out_h, in_o, out_o, sem_ih, sem_oh, sem_io, sem_oo):
  wid = lax.axis_index("s") * NC + lax.axis_index("c")

  iota = lax.iota(jnp.int32, L)
  cols = [jnp.full((L,), d, jnp.int32) for d in range(D)]

  def start_in(src_hbm, buf, sem, c):
    pltpu.async_copy(src_hbm.at[:, pl.ds(c * VC, VC)], buf, sem)

  def wait_in(src_hbm, buf, sem):
    pltpu.make_async_copy(src_hbm.at[:, pl.ds(0, VC)], buf, sem).wait()

  def start_out(buf, dst_hbm, sem, c):
    pltpu.async_copy(buf.at[:, pl.ds(0, D)], dst_hbm.at[pl.ds(c * VC, VC)],
                     sem)

  def wait_out(buf, dst_hbm, sem):
    pltpu.make_async_copy(buf.at[:, pl.ds(0, D)], dst_hbm.at[pl.ds(0, VC)],
                          sem).wait()

  def transpose(inb, outb):
    def jgroup(g, _):
      j = g * L
      rows = j + iota
      hvs = [inb[d, pl.ds(j, L)] for d in range(D)]
      for d in range(D):
        plsc.store_scatter(outb, [rows, cols[d]], hvs[d])
      return 0

    lax.fori_loop(0, VC // L, jgroup, 0)

  @pl.when(wid < NCHUNK)
  def _():
    start_in(wht_hbm, in_h, sem_ih, wid)

  def body(k, _):
    c = wid + NW * k
    v = c < NCHUNK
    vp = jnp.logical_and(k > 0, c - NW < NCHUNK)
    vn = c + NW < NCHUNK

    @pl.when(v)
    def _():
      wait_in(wht_hbm, in_h, sem_ih)
      start_in(wot_hbm, in_o, sem_io, c)

    @pl.when(vp)
    def _():
      wait_out(out_h, wh_out, sem_oh)

    @pl.when(v)
    def _():
      transpose(in_h, out_h)
      start_out(out_h, wh_out, sem_oh, c)
      wait_in(wot_hbm, in_o, sem_io)

    @pl.when(vn)
    def _():
      start_in(wht_hbm, in_h, sem_ih, c + NW)

    @pl.when(vp)
    def _():
      wait_out(out_o, wo_out, sem_oo)

    @pl.when(v)
    def _():
      transpose(in_o, out_o)
      start_out(out_o, wo_out, sem_oo, c)

    return 0

  lax.fori_loop(0, NK, body, 0)

  @pl.when(wid + NW * (NK - 1) < NCHUNK)
  def _():
    wait_out(out_h, wh_out, sem_oh)
    wait_out(out_o, wo_out, sem_oo)


@jax.jit
def _sc_transpose(wht, wot):
  mesh = plsc.VectorSubcoreMesh(core_axis_name="c", subcore_axis_name="s")
  return pl.kernel(
      _tr_body,
      out_type=(
          jax.ShapeDtypeStruct((VOCAB, D), jnp.float32),
          jax.ShapeDtypeStruct((VOCAB, D), jnp.float32),
      ),
      mesh=mesh,
      compiler_params=pltpu.CompilerParams(
          needs_layout_passes=False, use_tc_tiling_on_sc=False),
      scratch_types=[
          pltpu.VMEM((D, VC), jnp.float32),
          pltpu.VMEM((VC, DP), jnp.float32),
          pltpu.VMEM((D, VC), jnp.float32),
          pltpu.VMEM((VC, DP), jnp.float32),
          pltpu.SemaphoreType.DMA,
          pltpu.SemaphoreType.DMA,
          pltpu.SemaphoreType.DMA,
          pltpu.SemaphoreType.DMA,
      ],
  )(wht, wot)


def _sc_body(x_hbm, y_hbm, negt_hbm, wh_hbm, wo_hbm, pos_out, neg_out,
             xi, yi, ni, h, t, a, pos_v, neg_v, sem):
  wid = lax.axis_index("s") * NC + lax.axis_index("c")
  base = wid * BPW

  zero = jnp.zeros((L,), jnp.float32)

  def zrow(i, _):
    a[i, pl.ds(0, L)] = zero
    a[i, pl.ds(L, L)] = zero
    return 0

  lax.fori_loop(0, BPW, zrow, 0)

  pltpu.sync_copy(x_hbm.at[pl.ds(base, BPW)], xi)
  pltpu.sync_copy(y_hbm.at[pl.ds(base, BPW)], yi)
  pltpu.sync_copy(negt_hbm.at[:, pl.ds(base, BPW)], ni)

  cps = [pltpu.async_copy(wh_hbm.at[xi], h, sem),
         pltpu.async_copy(wo_hbm.at[yi], t, sem)]
  cps += [pltpu.async_copy(wo_hbm.at[ni.at[n]], a, sem, add=True)
          for n in range(N_NEG)]
  for cp in cps:
    cp.wait()

  def row16(i, _):
    b = i * L
    rows = b + lax.iota(jnp.int32, L)
    pacc = jnp.zeros((L,), jnp.float32)
    nacc = jnp.zeros((L,), jnp.float32)
    for d in range(D):
      cols = jnp.full((L,), d, jnp.int32)
      hv = plsc.load_gather(h, [rows, cols])
      pacc = pacc + plsc.load_gather(t, [rows, cols]) * hv
      nacc = nacc + plsc.load_gather(a, [rows, cols]) * hv
    pos_v[pl.ds(b, L)] = pacc
    neg_v[pl.ds(b, L)] = nacc
    return 0

  lax.fori_loop(0, BPW // L, row16, 0)

  pltpu.sync_copy(pos_v, pos_out.at[pl.ds(base, BPW)])
  pltpu.sync_copy(neg_v, neg_out.at[pl.ds(base, BPW)])


@jax.jit
def _sc_scores(x, y, neg_t, w_hidden, w_output):
  mesh = plsc.VectorSubcoreMesh(core_axis_name="c", subcore_axis_name="s")
  return pl.kernel(
      _sc_body,
      out_type=(
          jax.ShapeDtypeStruct((B,), jnp.float32),
          jax.ShapeDtypeStruct((B,), jnp.float32),
      ),
      mesh=mesh,
      compiler_params=pltpu.CompilerParams(
          needs_layout_passes=False, use_tc_tiling_on_sc=False),
      scratch_types=[
          pltpu.VMEM((BPW,), jnp.int32),
          pltpu.VMEM((BPW,), jnp.int32),
          pltpu.VMEM((N_NEG, BPW), jnp.int32),
          pltpu.VMEM((BPW, D), jnp.float32),
          pltpu.VMEM((BPW, D), jnp.float32),
          pltpu.VMEM((BPW, D), jnp.float32),
          pltpu.VMEM((BPW,), jnp.float32),
          pltpu.VMEM((BPW,), jnp.float32),
          pltpu.SemaphoreType.DMA,
      ],
  )(x, y, neg_t, w_hidden, w_output)


def _log_sigmoid(z):
  return jnp.minimum(z, 0.0) - jnp.log1p(jnp.exp(-jnp.abs(z)))


def _loss_body(pos_ref, neg_ref, out_ref):
  pos = pos_ref[...]
  neg = -neg_ref[...]
  loss = _log_sigmoid(pos) + _log_sigmoid(neg)
  out_ref[0, 0] = -jnp.sum(loss) / B


@jax.jit
def _tc_loss(pos, neg):
  out = pl.pallas_call(
      _loss_body,
      out_shape=jax.ShapeDtypeStruct((1, 1), jnp.float32),
      out_specs=pl.BlockSpec(memory_space=pltpu.SMEM),
  )(pos.reshape(128, 128), neg.reshape(128, 128))
  return out[0, 0]


def kernel(x, y, negative_batch, W_hidden, W_output):
  xf = x.reshape(B)
  yf = y.reshape(B)
  neg_t = negative_batch.T
  wh_rm, wo_rm = _sc_transpose(W_hidden.T, W_output.T)
  pos, negdot = _sc_scores(xf, yf, neg_t, wh_rm, wo_rm)
  return _tc_loss(pos, negdot)


# speedup vs baseline: 6.0820x; 6.0820x over previous
import functools

import jax
import jax.numpy as jnp
from jax import lax
from jax.experimental import pallas as pl
from jax.experimental.pallas import tpu as pltpu
from jax.experimental.pallas import tpu_sc as plsc

VOCAB = 1000000
B = 16384
D = 32
N_NEG = 20
L = 16
NC = 2
NS = 16
NW = NC * NS
BPW = B // NW

def _sc_body(x_hbm, y_hbm, negt_hbm, wh_hbm, wo_hbm, pos_out, neg_out,
             xi, yi, ni, h, t, a, pos_v, neg_v, sem):
  wid = lax.axis_index("s") * NC + lax.axis_index("c")
  base = wid * BPW

  zero = jnp.zeros((L,), jnp.float32)

  def zrow(i, _):
    a[i, pl.ds(0, L)] = zero
    a[i, pl.ds(L, L)] = zero
    return 0

  lax.fori_loop(0, BPW, zrow, 0)

  pltpu.sync_copy(x_hbm.at[pl.ds(base, BPW)], xi)
  pltpu.sync_copy(y_hbm.at[pl.ds(base, BPW)], yi)
  pltpu.sync_copy(negt_hbm.at[:, pl.ds(base, BPW)], ni)

  cps = [pltpu.async_copy(wh_hbm.at[xi], h, sem),
         pltpu.async_copy(wo_hbm.at[yi], t, sem)]
  cps += [pltpu.async_copy(wo_hbm.at[ni.at[n]], a, sem, add=True)
          for n in range(N_NEG)]
  for cp in cps:
    cp.wait()

  def row16(i, _):
    b = i * L
    rows = b + lax.iota(jnp.int32, L)
    pacc = jnp.zeros((L,), jnp.float32)
    nacc = jnp.zeros((L,), jnp.float32)
    for d in range(D):
      cols = jnp.full((L,), d, jnp.int32)
      hv = plsc.load_gather(h, [rows, cols])
      pacc = pacc + plsc.load_gather(t, [rows, cols]) * hv
      nacc = nacc + plsc.load_gather(a, [rows, cols]) * hv
    pos_v[pl.ds(b, L)] = pacc
    neg_v[pl.ds(b, L)] = nacc
    return 0

  lax.fori_loop(0, BPW // L, row16, 0)

  pltpu.sync_copy(pos_v, pos_out.at[pl.ds(base, BPW)])
  pltpu.sync_copy(neg_v, neg_out.at[pl.ds(base, BPW)])


@jax.jit
def _sc_scores(x, y, neg_t, w_hidden, w_output):
  mesh = plsc.VectorSubcoreMesh(core_axis_name="c", subcore_axis_name="s")
  return pl.kernel(
      _sc_body,
      out_type=(
          jax.ShapeDtypeStruct((B,), jnp.float32),
          jax.ShapeDtypeStruct((B,), jnp.float32),
      ),
      mesh=mesh,
      compiler_params=pltpu.CompilerParams(
          needs_layout_passes=False, use_tc_tiling_on_sc=False),
      scratch_types=[
          pltpu.VMEM((BPW,), jnp.int32),
          pltpu.VMEM((BPW,), jnp.int32),
          pltpu.VMEM((N_NEG, BPW), jnp.int32),
          pltpu.VMEM((BPW, D), jnp.float32),
          pltpu.VMEM((BPW, D), jnp.float32),
          pltpu.VMEM((BPW, D), jnp.float32),
          pltpu.VMEM((BPW,), jnp.float32),
          pltpu.VMEM((BPW,), jnp.float32),
          pltpu.SemaphoreType.DMA,
      ],
  )(x, y, neg_t, w_hidden, w_output)


def _log_sigmoid(z):
  return jnp.minimum(z, 0.0) - jnp.log1p(jnp.exp(-jnp.abs(z)))


def _loss_body(pos_ref, neg_ref, out_ref):
  pos = pos_ref[...]
  neg = -neg_ref[...]
  loss = _log_sigmoid(pos) + _log_sigmoid(neg)
  out_ref[0, 0] = -jnp.sum(loss) / B


@jax.jit
def _tc_loss(pos, neg):
  out = pl.pallas_call(
      _loss_body,
      out_shape=jax.ShapeDtypeStruct((1, 1), jnp.float32),
      out_specs=pl.BlockSpec(memory_space=pltpu.SMEM),
  )(pos.reshape(128, 128), neg.reshape(128, 128))
  return out[0, 0]


def kernel(x, y, negative_batch, W_hidden, W_output):
  xf = x.reshape(B)
  yf = y.reshape(B)
  neg_t = negative_batch.T
  pos, negdot = _sc_scores(xf, yf, neg_t, W_hidden, W_output)
  return _tc_loss(pos, negdot)
